# baseline (speedup 1.0000x reference)
import functools

import jax
import jax.numpy as jnp
from jax import lax
from jax.experimental import pallas as pl
from jax.experimental.pallas import tpu as pltpu
from jax.experimental.pallas import tpu_sc as plsc

NC, NS, L = 2, 16, 16
NW = NC * NS
B = 16384
D = 64
VOCAB = 1000000
K = 5
BPW = B // NW
CH = 128
NCHUNK = BPW // CH
GRP = CH // L
DU = 16


def _logsig(x):
    e = jnp.exp(-jnp.abs(x))
    t = e / (2.0 + e)
    t2 = t * t
    p = 1.0 / 9.0
    for c in (1.0 / 7.0, 1.0 / 5.0, 1.0 / 3.0, 1.0):
        p = p * t2 + c
    return jnp.minimum(x, 0.0) - 2.0 * t * p


_mesh = plsc.VectorSubcoreMesh(core_axis_name="c", subcore_axis_name="s")


@functools.partial(
    pl.kernel,
    mesh=_mesh,
    compiler_params=pltpu.CompilerParams(
        needs_layout_passes=False, use_tc_tiling_on_sc=False),
    out_type=jax.ShapeDtypeStruct((NW // 8, 8 * L), jnp.float32),
    scratch_types=[
        pltpu.VMEM((BPW,), jnp.int32),
        pltpu.VMEM((BPW,), jnp.int32),
        pltpu.VMEM((BPW * K,), jnp.int32),
        pltpu.VMEM((BPW,), jnp.int32),
        pltpu.VMEM((CH, D), jnp.float32),
        pltpu.VMEM((CH, D), jnp.float32),
        pltpu.VMEM((CH, D), jnp.float32),
        pltpu.VMEM((CH, D), jnp.float32),
        pltpu.VMEM((CH * K, D), jnp.float32),
        pltpu.VMEM((CH * K, D), jnp.float32),
        pltpu.VMEM((L,), jnp.float32),
        pltpu.SemaphoreType.DMA,
        pltpu.SemaphoreType.DMA,
    ],
)
def _sc_loss(center_hbm, pos_hbm, negf_hbm, inemb_hbm, outemb_hbm,
             out_hbm, ci, pi, ni, cih, rc0, rc1, rp0, rp1,
             rn0, rn1, accv, sem0, sem1):
    rc = (rc0, rc1)
    rp = (rp0, rp1)
    rn = (rn0, rn1)
    wid = lax.axis_index("s") * NC + lax.axis_index("c")
    base = wid * BPW

    pltpu.sync_copy(center_hbm.at[pl.ds(base, BPW)], ci)
    pltpu.sync_copy(pos_hbm.at[pl.ds(base, BPW)], pi)
    pltpu.sync_copy(negf_hbm.at[pl.ds(base * K, BPW * K)], ni)

    def _remap(src, dst, n):
        def hbody(i, _):
            v = src[pl.ds(i * L, L)]
            pair = (lax.shift_left(lax.shift_right_logical(v, SH_I), SH_O)
                    + (v & MSK))
            half = lax.shift_right_logical(v, SH_O) & 1
            dst[pl.ds(i * L, L)] = lax.shift_left(pair, 1) + half
            return 0
        lax.fori_loop(0, n // L, hbody, 0)

    _remap(ci, cih, BPW)

    sems = (sem0, sem1)

    def issue(c):
        s = c % 2
        hs = [
            pltpu.async_copy(inemb_hbm.at[cih.at[pl.ds(c * CH, CH)]],
                             rc[s], sems[s]),
            pltpu.async_copy(outemb_hbm.at[pi.at[pl.ds(c * CH, CH)]],
                             rp[s], sems[s]),
        ]
        for j in range(K):
            hs.append(pltpu.async_copy(
                outemb_hbm.at[ni.at[pl.ds(c * CH * K + j * CH, CH)]],
                rn[s].at[pl.ds(j * CH, CH)], sems[s]))
        return hs

    handles = {0: issue(0)}
    acc = jnp.zeros((L,), jnp.float32)

    for c in range(NCHUNK):
        if c + 1 < NCHUNK:
            handles[c + 1] = issue(c + 1)
        for h in handles.pop(c):
            h.wait()
        s = c % 2
        rc_s, rp_s, rn_s = rc[s], rp[s], rn[s]

        def gbody(g, acc):
            rows = g * L + lax.iota(jnp.int32, L)
            rows5 = rows * K
            rows5k = [rows5 + k for k in range(K)]

            def dbody(dbase, carry):
                pos, n0, n1, n2, n3, n4 = carry
                off = jnp.full((L,), dbase * DU, jnp.int32)
                ns = [n0, n1, n2, n3, n4]
                for dd in range(DU):
                    col = off + dd
                    cd = plsc.load_gather(rc_s, [rows, col])
                    pd = plsc.load_gather(rp_s, [rows, col])
                    pos = pos + cd * pd
                    for k in range(K):
                        nd = plsc.load_gather(rn_s, [rows5k[k], col])
                        ns[k] = ns[k] + cd * nd
                return (pos, ns[0], ns[1], ns[2], ns[3], ns[4])

            z = jnp.zeros((L,), jnp.float32)
            pos, n0, n1, n2, n3, n4 = lax.fori_loop(
                0, D // DU, dbody, (z, z, z, z, z, z))
            tot = _logsig(pos)
            for nk in (n0, n1, n2, n3, n4):
                tot = tot + _logsig(-nk)
            return acc - tot

        acc = lax.fori_loop(0, GRP, gbody, acc)

    accv[...] = acc
    pltpu.sync_copy(accv, out_hbm.at[wid // 8, pl.ds((wid % 8) * L, L)])


TRBI = 16384
TRO = TRBI // 2
NBLK = (VOCAB + TRBI - 1) // TRBI
PROWS = NBLK * TRO
SH_I = TRBI.bit_length() - 1
SH_O = TRO.bit_length() - 1
MSK = TRO - 1


def _transpose_body(a_ref, oa_ref):
    oa_ref[...] = jnp.concatenate(
        [a_ref[:, 0:TRO], a_ref[:, TRO:TRBI]], axis=0).T


_transpose = pl.pallas_call(
    _transpose_body,
    grid=(NBLK,),
    in_specs=[pl.BlockSpec((D, TRBI), lambda i: (0, i))],
    out_specs=pl.BlockSpec((TRO, 2 * D), lambda i: (i, 0)),
    out_shape=jax.ShapeDtypeStruct((PROWS, 2 * D), jnp.float32),
)


def _sum_body(x_ref, o_ref):
    o_ref[...] = jnp.full((1, 1), jnp.sum(x_ref[...]) * (1.0 / B),
                          jnp.float32)


_sum = pl.pallas_call(
    _sum_body,
    out_shape=jax.ShapeDtypeStruct((1, 1), jnp.float32),
)


def kernel(center, pos_context, neg_context, in_embed, out_embed):
    center = center.astype(jnp.int32)
    pos_context = pos_context.astype(jnp.int32)
    neg_flat = neg_context.astype(jnp.int32).reshape(-1)
    r_in = _transpose(in_embed.T).reshape(2 * PROWS, D)
    partials = _sc_loss(center, pos_context, neg_flat, r_in, out_embed)
    return _sum(partials)[0, 0]


# speedup vs baseline: 2.6217x; 2.6217x over previous
import functools

import jax
import jax.numpy as jnp
from jax import lax
from jax.experimental import pallas as pl
from jax.experimental.pallas import tpu as pltpu
from jax.experimental.pallas import tpu_sc as plsc

NC, NS, L = 2, 16, 16
NW = NC * NS
B = 16384
D = 64
VOCAB = 1000000
K = 5
BPW = B // NW
CH = 128
NCHUNK = BPW // CH
GRP = CH // L
DU = 16


def _logsig(x):
    e = jnp.exp(-jnp.abs(x))
    t = e / (2.0 + e)
    t2 = t * t
    p = 1.0 / 9.0
    for c in (1.0 / 7.0, 1.0 / 5.0, 1.0 / 3.0, 1.0):
        p = p * t2 + c
    return jnp.minimum(x, 0.0) - 2.0 * t * p


_mesh = plsc.VectorSubcoreMesh(core_axis_name="c", subcore_axis_name="s")


@functools.partial(
    pl.kernel,
    mesh=_mesh,
    compiler_params=pltpu.CompilerParams(
        needs_layout_passes=False, use_tc_tiling_on_sc=False),
    out_type=jax.ShapeDtypeStruct((NW // 8, 8 * L), jnp.float32),
    scratch_types=[
        pltpu.VMEM((BPW,), jnp.int32),
        pltpu.VMEM((BPW,), jnp.int32),
        pltpu.VMEM((BPW * K,), jnp.int32),
        pltpu.VMEM((BPW,), jnp.int32),
        pltpu.VMEM((BPW,), jnp.int32),
        pltpu.VMEM((BPW * K,), jnp.int32),
        pltpu.VMEM((CH, D // 2), jnp.int32),
        pltpu.VMEM((CH, D // 2), jnp.int32),
        pltpu.VMEM((CH, D // 2), jnp.int32),
        pltpu.VMEM((CH, D // 2), jnp.int32),
        pltpu.VMEM((CH * K, D // 2), jnp.int32),
        pltpu.VMEM((CH * K, D // 2), jnp.int32),
        pltpu.VMEM((L,), jnp.float32),
        pltpu.SemaphoreType.DMA,
        pltpu.SemaphoreType.DMA,
    ],
)
def _sc_loss(center_hbm, pos_hbm, negf_hbm, inemb_hbm, outemb_hbm,
             out_hbm, ci, pi, ni, cih, pih, nih, rc0, rc1, rp0, rp1,
             rn0, rn1, accv, sem0, sem1):
    rc = (rc0, rc1)
    rp = (rp0, rp1)
    rn = (rn0, rn1)
    wid = lax.axis_index("s") * NC + lax.axis_index("c")
    base = wid * BPW

    pltpu.sync_copy(center_hbm.at[pl.ds(base, BPW)], ci)
    pltpu.sync_copy(pos_hbm.at[pl.ds(base, BPW)], pi)
    pltpu.sync_copy(negf_hbm.at[pl.ds(base * K, BPW * K)], ni)

    def _remap(src, dst, n):
        def hbody(i, _):
            v = src[pl.ds(i * L, L)]
            quad = (lax.shift_left(lax.shift_right_logical(v, SH_I), T4SH)
                    + (v & M4))
            quarter = lax.shift_right_logical(v, T4SH) & 3
            dst[pl.ds(i * L, L)] = lax.shift_left(quad, 2) + quarter
            return 0
        lax.fori_loop(0, n // L, hbody, 0)

    _remap(ci, cih, BPW)
    _remap(pi, pih, BPW)
    _remap(ni, nih, BPW * K)

    sems = (sem0, sem1)

    def issue(c):
        s = c % 2
        hs = [
            pltpu.async_copy(inemb_hbm.at[cih.at[pl.ds(c * CH, CH)]],
                             rc[s], sems[s]),
            pltpu.async_copy(outemb_hbm.at[pih.at[pl.ds(c * CH, CH)]],
                             rp[s], sems[s]),
        ]
        for j in range(K):
            hs.append(pltpu.async_copy(
                outemb_hbm.at[nih.at[pl.ds(c * CH * K + j * CH, CH)]],
                rn[s].at[pl.ds(j * CH, CH)], sems[s]))
        return hs

    handles = {0: issue(0)}
    acc = jnp.zeros((L,), jnp.float32)

    for c in range(NCHUNK):
        if c + 1 < NCHUNK:
            handles[c + 1] = issue(c + 1)
        for h in handles.pop(c):
            h.wait()
        s = c % 2
        rc_s, rp_s, rn_s = rc[s], rp[s], rn[s]

        def gbody(g, acc):
            rows = g * L + lax.iota(jnp.int32, L)
            rows5 = rows * K
            rows5k = [rows5 + k for k in range(K)]

            himsk = jnp.full((L,), -65536, jnp.int32)

            def _unpk(x):
                lo = plsc.bitcast(lax.shift_left(x, 16), jnp.float32)
                hi = plsc.bitcast(x & himsk, jnp.float32)
                return lo, hi

            def dbody(dbase, carry):
                pos, n0, n1, n2, n3, n4 = carry
                off = jnp.full((L,), dbase * DU, jnp.int32)
                ns = [n0, n1, n2, n3, n4]
                for dd in range(DU):
                    col = off + dd
                    cl, ch = _unpk(plsc.load_gather(rc_s, [rows, col]))
                    pl_, ph = _unpk(plsc.load_gather(rp_s, [rows, col]))
                    pos = pos + cl * pl_ + ch * ph
                    for k in range(K):
                        nl, nh = _unpk(
                            plsc.load_gather(rn_s, [rows5k[k], col]))
                        ns[k] = ns[k] + cl * nl + ch * nh
                return (pos, ns[0], ns[1], ns[2], ns[3], ns[4])

            z = jnp.zeros((L,), jnp.float32)
            pos, n0, n1, n2, n3, n4 = lax.fori_loop(
                0, (D // 2) // DU, dbody, (z, z, z, z, z, z))
            tot = _logsig(pos)
            for nk in (n0, n1, n2, n3, n4):
                tot = tot + _logsig(-nk)
            return acc - tot

        acc = lax.fori_loop(0, GRP, gbody, acc)

    accv[...] = acc
    pltpu.sync_copy(accv, out_hbm.at[wid // 8, pl.ds((wid % 8) * L, L)])


TRBI = 16384
NBLK = (VOCAB + TRBI - 1) // TRBI
SH_I = TRBI.bit_length() - 1
T4SH = (TRBI // 4).bit_length() - 1
M4 = TRBI // 4 - 1


T4 = TRBI // 4
QROWS = NBLK * T4
_HIMSK = -65536


def _transpose_body(a_ref, b_ref, oa_ref, ob_ref):
    for x, o in ((a_ref, oa_ref), (b_ref, ob_ref)):
        lo = x[0:D // 2, :]
        hi = x[D // 2:D, :]
        ilo = lax.shift_right_logical(
            lax.bitcast_convert_type(
                lo.astype(jnp.bfloat16).astype(jnp.float32), jnp.int32),
            16)
        ihi = lax.bitcast_convert_type(
            hi.astype(jnp.bfloat16).astype(jnp.float32), jnp.int32) & _HIMSK
        p = ihi | ilo
        o[...] = jnp.concatenate(
            [p[:, 0:T4], p[:, T4:2 * T4],
             p[:, 2 * T4:3 * T4], p[:, 3 * T4:4 * T4]], axis=0).T


_transpose = pl.pallas_call(
    _transpose_body,
    grid=(NBLK,),
    in_specs=[
        pl.BlockSpec((D, TRBI), lambda i: (0, i)),
        pl.BlockSpec((D, TRBI), lambda i: (0, i)),
    ],
    out_specs=[
        pl.BlockSpec((T4, 4 * (D // 2)), lambda i: (i, 0)),
        pl.BlockSpec((T4, 4 * (D // 2)), lambda i: (i, 0)),
    ],
    out_shape=[
        jax.ShapeDtypeStruct((QROWS, 4 * (D // 2)), jnp.int32),
        jax.ShapeDtypeStruct((QROWS, 4 * (D // 2)), jnp.int32),
    ],
)


def _sum_body(x_ref, o_ref):
    o_ref[...] = jnp.full((1, 1), jnp.sum(x_ref[...]) * (1.0 / B),
                          jnp.float32)


_sum = pl.pallas_call(
    _sum_body,
    out_shape=jax.ShapeDtypeStruct((1, 1), jnp.float32),
)


def kernel(center, pos_context, neg_context, in_embed, out_embed):
    center = center.astype(jnp.int32)
    pos_context = pos_context.astype(jnp.int32)
    neg_flat = neg_context.astype(jnp.int32).reshape(-1)
    r_in, r_out = _transpose(in_embed.T, out_embed.T)
    r_in = r_in.reshape(4 * QROWS, D // 2)
    r_out = r_out.reshape(4 * QROWS, D // 2)
    partials = _sc_loss(center, pos_context, neg_flat, r_in, r_out)
    return _sum(partials)[0, 0]


# speedup vs baseline: 2.6863x; 1.0247x over previous
import functools

import jax
import jax.numpy as jnp
from jax import lax
from jax.experimental import pallas as pl
from jax.experimental.pallas import tpu as pltpu
from jax.experimental.pallas import tpu_sc as plsc

NC, NS, L = 2, 16, 16
NW = NC * NS
B = 16384
D = 64
VOCAB = 1000000
K = 5
BPW = B // NW
CH = 128
NCHUNK = BPW // CH
GRP = CH // L
DU = 16


def _logsig(x):
    e = jnp.exp(-jnp.abs(x))
    t = e / (2.0 + e)
    t2 = t * t
    p = 1.0 / 9.0
    for c in (1.0 / 7.0, 1.0 / 5.0, 1.0 / 3.0, 1.0):
        p = p * t2 + c
    return jnp.minimum(x, 0.0) - 2.0 * t * p


_mesh = plsc.VectorSubcoreMesh(core_axis_name="c", subcore_axis_name="s")


@functools.partial(
    pl.kernel,
    mesh=_mesh,
    compiler_params=pltpu.CompilerParams(
        needs_layout_passes=False, use_tc_tiling_on_sc=False),
    out_type=jax.ShapeDtypeStruct((NW // 8, 8 * L), jnp.float32),
    scratch_types=[
        pltpu.VMEM((BPW,), jnp.int32),
        pltpu.VMEM((BPW,), jnp.int32),
        pltpu.VMEM((BPW * K,), jnp.int32),
        pltpu.VMEM((BPW,), jnp.int32),
        pltpu.VMEM((BPW,), jnp.int32),
        pltpu.VMEM((BPW * K,), jnp.int32),
        pltpu.VMEM((CH, D // 2), jnp.int32),
        pltpu.VMEM((CH, D // 2), jnp.int32),
        pltpu.VMEM((CH, D // 2), jnp.int32),
        pltpu.VMEM((CH, D // 2), jnp.int32),
        pltpu.VMEM((CH * K, D // 2), jnp.int32),
        pltpu.VMEM((CH * K, D // 2), jnp.int32),
        pltpu.VMEM((L,), jnp.float32),
        pltpu.SemaphoreType.DMA,
        pltpu.SemaphoreType.DMA,
    ],
)
def _sc_loss(center_hbm, pos_hbm, negf_hbm, inemb_hbm, outemb_hbm,
             out_hbm, ci, pi, ni, cih, pih, nih, rc0, rc1, rp0, rp1,
             rn0, rn1, accv, sem0, sem1):
    rc = (rc0, rc1)
    rp = (rp0, rp1)
    rn = (rn0, rn1)
    wid = lax.axis_index("s") * NC + lax.axis_index("c")
    base = wid * BPW

    pltpu.sync_copy(center_hbm.at[pl.ds(base, BPW)], ci)
    pltpu.sync_copy(pos_hbm.at[pl.ds(base, BPW)], pi)
    for k in range(K):
        pltpu.sync_copy(negf_hbm.at[pl.ds(k * B + base, BPW)],
                        ni.at[pl.ds(k * BPW, BPW)])

    def _remap(src, dst, n):
        def hbody(i, _):
            v = src[pl.ds(i * L, L)]
            quad = (lax.shift_left(lax.shift_right_logical(v, SH_I), T4SH)
                    + (v & M4))
            quarter = lax.shift_right_logical(v, T4SH) & 3
            dst[pl.ds(i * L, L)] = lax.shift_left(quad, 2) + quarter
            return 0
        lax.fori_loop(0, n // L, hbody, 0)

    _remap(ci, cih, BPW)
    _remap(pi, pih, BPW)
    _remap(ni, nih, BPW * K)

    sems = (sem0, sem1)

    def issue(c):
        s = c % 2
        hs = [
            pltpu.async_copy(inemb_hbm.at[cih.at[pl.ds(c * CH, CH)]],
                             rc[s], sems[s]),
            pltpu.async_copy(outemb_hbm.at[pih.at[pl.ds(c * CH, CH)]],
                             rp[s], sems[s]),
        ]
        for j in range(K):
            hs.append(pltpu.async_copy(
                outemb_hbm.at[nih.at[pl.ds(j * BPW + c * CH, CH)]],
                rn[s].at[pl.ds(j * CH, CH)], sems[s]))
        return hs

    handles = {0: issue(0)}
    acc = jnp.zeros((L,), jnp.float32)

    for c in range(NCHUNK):
        if c + 1 < NCHUNK:
            handles[c + 1] = issue(c + 1)
        for h in handles.pop(c):
            h.wait()
        s = c % 2
        rc_s, rp_s, rn_s = rc[s], rp[s], rn[s]

        def gbody(g, acc):
            rows = g * L + lax.iota(jnp.int32, L)
            rows5k = [rows + k * CH for k in range(K)]

            himsk = jnp.full((L,), -65536, jnp.int32)

            def _unpk(x):
                lo = plsc.bitcast(lax.shift_left(x, 16), jnp.float32)
                hi = plsc.bitcast(x & himsk, jnp.float32)
                return lo, hi

            def dbody(dbase, carry):
                pos, n0, n1, n2, n3, n4 = carry
                off = jnp.full((L,), dbase * DU, jnp.int32)
                ns = [n0, n1, n2, n3, n4]
                for dd in range(DU):
                    col = off + dd
                    cl, ch = _unpk(plsc.load_gather(rc_s, [rows, col]))
                    pl_, ph = _unpk(plsc.load_gather(rp_s, [rows, col]))
                    pos = pos + cl * pl_ + ch * ph
                    for k in range(K):
                        nl, nh = _unpk(
                            plsc.load_gather(rn_s, [rows5k[k], col]))
                        ns[k] = ns[k] + cl * nl + ch * nh
                return (pos, ns[0], ns[1], ns[2], ns[3], ns[4])

            z = jnp.zeros((L,), jnp.float32)
            pos, n0, n1, n2, n3, n4 = lax.fori_loop(
                0, (D // 2) // DU, dbody, (z, z, z, z, z, z))
            tot = _logsig(pos)
            for nk in (n0, n1, n2, n3, n4):
                tot = tot + _logsig(-nk)
            return acc - tot

        acc = lax.fori_loop(0, GRP, gbody, acc)

    accv[...] = acc
    pltpu.sync_copy(accv, out_hbm.at[wid // 8, pl.ds((wid % 8) * L, L)])


TRBI = 16384
NBLK = (VOCAB + TRBI - 1) // TRBI
SH_I = TRBI.bit_length() - 1
T4SH = (TRBI // 4).bit_length() - 1
M4 = TRBI // 4 - 1


T4 = TRBI // 4
QROWS = NBLK * T4
_HIMSK = -65536


def _transpose_body(a_ref, b_ref, oa_ref, ob_ref):
    for x, o in ((a_ref, oa_ref), (b_ref, ob_ref)):
        lo = x[0:D // 2, :]
        hi = x[D // 2:D, :]
        ilo = lax.shift_right_logical(
            lax.bitcast_convert_type(
                lo.astype(jnp.bfloat16).astype(jnp.float32), jnp.int32),
            16)
        ihi = lax.bitcast_convert_type(
            hi.astype(jnp.bfloat16).astype(jnp.float32), jnp.int32) & _HIMSK
        p = ihi | ilo
        o[...] = jnp.concatenate(
            [p[:, 0:T4], p[:, T4:2 * T4],
             p[:, 2 * T4:3 * T4], p[:, 3 * T4:4 * T4]], axis=0).T


_transpose = pl.pallas_call(
    _transpose_body,
    grid=(NBLK,),
    in_specs=[
        pl.BlockSpec((D, TRBI), lambda i: (0, i)),
        pl.BlockSpec((D, TRBI), lambda i: (0, i)),
    ],
    out_specs=[
        pl.BlockSpec((T4, 4 * (D // 2)), lambda i: (i, 0)),
        pl.BlockSpec((T4, 4 * (D // 2)), lambda i: (i, 0)),
    ],
    out_shape=[
        jax.ShapeDtypeStruct((QROWS, 4 * (D // 2)), jnp.int32),
        jax.ShapeDtypeStruct((QROWS, 4 * (D // 2)), jnp.int32),
    ],
)


def _sum_body(x_ref, o_ref):
    o_ref[...] = jnp.full((1, 1), jnp.sum(x_ref[...]) * (1.0 / B),
                          jnp.float32)


_sum = pl.pallas_call(
    _sum_body,
    out_shape=jax.ShapeDtypeStruct((1, 1), jnp.float32),
)


def kernel(center, pos_context, neg_context, in_embed, out_embed):
    center = center.astype(jnp.int32)
    pos_context = pos_context.astype(jnp.int32)
    neg_flat = neg_context.astype(jnp.int32).T.reshape(-1)
    r_in, r_out = _transpose(in_embed.T, out_embed.T)
    r_in = r_in.reshape(4 * QROWS, D // 2)
    r_out = r_out.reshape(4 * QROWS, D // 2)
    partials = _sc_loss(center, pos_context, neg_flat, r_in, r_out)
    return _sum(partials)[0, 0]
